# Initial kernel scaffold; baseline (speedup 1.0000x reference)
import jax
import jax.numpy as jnp
from jax import lax
from jax.experimental import pallas as pl

_H = 512
_W = 512
_R = 1024
_C = 256
_K = 1024


def _partner(x, j):
    n, c = x.shape
    xr = x.reshape(n // (2 * j), 2, j, c)
    return jnp.concatenate([xr[:, 1:2], xr[:, 0:1]], axis=1).reshape(n, c)


def _cmpex(arrs, j, desc, r):
    s, i = arrs[0], arrs[1]
    ps = _partner(s, j)
    pi = _partner(i, j)
    beats = (s > ps) | ((s == ps) & (i < pi))
    is_lo = (r & j) == 0
    keep = jnp.where(desc, is_lo == beats, is_lo != beats)
    out = [jnp.where(keep, s, ps), jnp.where(keep, i, pi)]
    for x in arrs[2:]:
        out.append(jnp.where(keep, x, _partner(x, j)))
    return out


def _sort_cols(arrs):
    n = arrs[0].shape[0]
    r = lax.broadcasted_iota(jnp.int32, arrs[0].shape, 0)
    kk = 2
    while kk <= n:
        desc = (r & kk) == 0
        j = kk // 2
        while j >= 1:
            arrs = _cmpex(arrs, j, desc, r)
            j //= 2
        kk *= 2
    return arrs


def _merge_tree(arrs):
    n = arrs[0].shape[0]
    while arrs[0].shape[1] > 1:
        w2 = arrs[0].shape[1] // 2
        a_list = [x[:, :w2] for x in arrs]
        b_list = [jnp.flip(x[:, w2:], axis=0) for x in arrs]
        beats = (a_list[0] > b_list[0]) | (
            (a_list[0] == b_list[0]) & (a_list[1] < b_list[1]))
        arrs = [jnp.where(beats, a, b) for a, b in zip(a_list, b_list)]
        r = lax.broadcasted_iota(jnp.int32, arrs[0].shape, 0)
        desc = jnp.full(arrs[0].shape, True)
        j = n // 2
        while j >= 1:
            arrs = _cmpex(arrs, j, desc, r)
            j //= 2
    return arrs


def _decode_body(pred_ref, coords_ref, scores_ref):
    s = pred_ref[0, 0].reshape(_R, _C)
    hh = pred_ref[0, 1].reshape(_R, _C)
    ww = pred_ref[0, 2].reshape(_R, _C)
    idx = (lax.broadcasted_iota(jnp.int32, (_R, _C), 0) * _C
           + lax.broadcasted_iota(jnp.int32, (_R, _C), 1))

    arrs = _sort_cols([s, idx, hh, ww])
    s1, i1, h1, w1 = _merge_tree(arrs)

    xg = (i1 & (_W - 1)).astype(jnp.float32)
    yg = (i1 >> 9).astype(jnp.float32)
    hb = jnp.maximum(h1, 1e-6) * _H
    wb = jnp.maximum(w1, 1e-6) * _W
    x1 = xg - wb * 0.5
    y1 = yg - hb * 0.5
    x2 = xg + wb * 0.5
    y2 = yg + hb * 0.5

    coords_ref[0, 0] = x1.reshape(8, 128)
    coords_ref[0, 1] = y1.reshape(8, 128)
    coords_ref[0, 2] = x2.reshape(8, 128)
    coords_ref[0, 3] = y2.reshape(8, 128)
    scores_ref[0] = s1.reshape(8, 128)


def kernel(preds):
    B = preds.shape[0]
    coords, scores = pl.pallas_call(
        _decode_body,
        grid=(B,),
        in_specs=[pl.BlockSpec((1, 3, _H, _W), lambda b: (b, 0, 0, 0))],
        out_specs=[
            pl.BlockSpec((1, 4, 8, 128), lambda b: (b, 0, 0, 0)),
            pl.BlockSpec((1, 8, 128), lambda b: (b, 0, 0)),
        ],
        out_shape=[
            jax.ShapeDtypeStruct((B, 4, 8, 128), jnp.float32),
            jax.ShapeDtypeStruct((B, 8, 128), jnp.float32),
        ],
    )(preds)
    boxes = coords.reshape(B, 4, _K).transpose(0, 2, 1)
    return boxes, scores.reshape(B, _K)


# speedup vs baseline: 20.2924x; 20.2924x over previous
import functools

import jax
import jax.numpy as jnp
from jax import lax
from jax.experimental import pallas as pl
from jax.experimental.pallas import tpu as pltpu
from jax.experimental.pallas import tpu_sc as plsc

_H = 512
_W = 512
_R = 1024
_C = 256
_K = 1024
_N = _H * _W


_roll = pltpu.roll


def _dyn_cmpex(s, i, j, desc):
    r = lax.broadcasted_iota(jnp.int32, s.shape, 0)
    is_lo = (r & j) == 0
    ps = jnp.where(is_lo, _roll(s, -j, 0), _roll(s, j, 0))
    pi = jnp.where(is_lo, _roll(i, -j, 0), _roll(i, j, 0))
    beats = (s > ps) | ((s == ps) & (i < pi))
    keep = is_lo ^ beats ^ desc
    return jnp.where(keep, s, ps), jnp.where(keep, i, pi)


def _coldesc(shape, w4):
    if w4 == 0:
        return lax.broadcasted_iota(jnp.int32, shape, 1) >= 0
    c = lax.broadcasted_iota(jnp.int32, shape, 1)
    return (c & w4) == 0


def _sort_cols(s, i, coldesc):
    r = lax.broadcasted_iota(jnp.int32, s.shape, 0)

    def phase(p, si):
        kk = jnp.left_shift(jnp.int32(1), p)
        desc = ~(((r & kk) == 0) ^ coldesc)

        def stage(t, si):
            j = lax.shift_right_logical(kk, t + 1)
            return _dyn_cmpex(si[0], si[1], j, desc)

        return lax.fori_loop(0, p, stage, si)

    return lax.fori_loop(1, 11, phase, (s, i))


def _merge_tree(s, i):
    while s.shape[1] > 1:
        w2 = s.shape[1] // 2
        sa, sb = s[:, :w2], s[:, w2:]
        ia, ib = i[:, :w2], i[:, w2:]
        beats = (sa > sb) | ((sa == sb) & (ia < ib))
        s = jnp.where(beats, sa, sb)
        i = jnp.where(beats, ia, ib)
        coldesc = _coldesc(s.shape, w2 // 2 if w2 > 1 else 0)

        def stage(t, si, _desc=coldesc):
            j = lax.shift_right_logical(jnp.int32(_R // 2), t)
            return _dyn_cmpex(si[0], si[1], j, _desc)

        s, i = lax.fori_loop(0, 10, stage, (s, i))
    return s, i


def _topk_body(score_ref, scores_ref, idx_ref):
    s = score_ref[0, 0].reshape(_R, _C)
    idx = (lax.broadcasted_iota(jnp.int32, (_R, _C), 0) * _C
           + lax.broadcasted_iota(jnp.int32, (_R, _C), 1))
    s, idx = _sort_cols(s, idx, _coldesc((_R, _C), _C // 2))
    s1, i1 = _merge_tree(s, idx)
    scores_ref[0] = s1.reshape(8, 128)
    idx_ref[0] = i1.reshape(8, 128)


def _run_topk(preds):
    B = preds.shape[0]
    return pl.pallas_call(
        _topk_body,
        grid=(B,),
        in_specs=[pl.BlockSpec((1, 1, _H, _W), lambda b: (b, 0, 0, 0))],
        out_specs=[
            pl.BlockSpec((1, 8, 128), lambda b: (b, 0, 0)),
            pl.BlockSpec((1, 8, 128), lambda b: (b, 0, 0)),
        ],
        out_shape=[
            jax.ShapeDtypeStruct((B, 8, 128), jnp.float32),
            jax.ShapeDtypeStruct((B, 8, 128), jnp.int32),
        ],
    )(preds[:, :1])


_NW = 32
_PER = (8 * _K) // _NW


def _box_body(hw_hbm, idx_hbm, out_hbm, idxv, gh, gw, hv, wv, c0, c1, c2, c3,
              sem):
    wid = lax.axis_index("s") * 2 + lax.axis_index("c")
    b = wid // 4
    q = wid % 4
    base = wid * _PER

    pltpu.sync_copy(idx_hbm.at[pl.ds(base, _PER)], idxv)

    hbase = (b * 2) * _N
    wbase = (b * 2 + 1) * _N
    for k in range(_PER // 16):
        iv = idxv[pl.ds(k * 16, 16)]
        gh[k // 8, pl.ds((k % 8) * 16, 16)] = iv + hbase
        gw[k // 8, pl.ds((k % 8) * 16, 16)] = iv + wbase

    cps = []
    for j in range(2):
        cps.append(pltpu.async_copy(
            hw_hbm.at[gh.at[j]], hv.at[pl.ds(j * 128, 128)], sem))
        cps.append(pltpu.async_copy(
            hw_hbm.at[gw.at[j]], wv.at[pl.ds(j * 128, 128)], sem))
    for cp in cps:
        cp.wait()

    for k in range(_PER // 16):
        sl = pl.ds(k * 16, 16)
        iv = idxv[sl]
        xg = (iv & (_W - 1)).astype(jnp.float32)
        yg = (iv >> 9).astype(jnp.float32)
        hb = jnp.maximum(hv[sl], 1e-6) * _H
        wb = jnp.maximum(wv[sl], 1e-6) * _W
        c0[sl] = xg - wb * 0.5
        c1[sl] = yg - hb * 0.5
        c2[sl] = xg + wb * 0.5
        c3[sl] = yg + hb * 0.5

    obase = b * 4 * _K + q * _PER
    pltpu.sync_copy(c0, out_hbm.at[pl.ds(obase, _PER)])
    pltpu.sync_copy(c1, out_hbm.at[pl.ds(obase + _K, _PER)])
    pltpu.sync_copy(c2, out_hbm.at[pl.ds(obase + 2 * _K, _PER)])
    pltpu.sync_copy(c3, out_hbm.at[pl.ds(obase + 3 * _K, _PER)])


def _run_boxes(preds, idx):
    hw_flat = preds[:, 1:3].reshape(-1)
    idx_flat = idx.reshape(-1)
    mesh = plsc.VectorSubcoreMesh(core_axis_name="c", subcore_axis_name="s")
    f = functools.partial(
        pl.kernel,
        mesh=mesh,
        out_type=jax.ShapeDtypeStruct((8 * 4 * _K,), jnp.float32),
        scratch_types=[
            pltpu.VMEM((_PER,), jnp.int32),
            pltpu.VMEM((2, 128), jnp.int32),
            pltpu.VMEM((2, 128), jnp.int32),
            pltpu.VMEM((_PER,), jnp.float32),
            pltpu.VMEM((_PER,), jnp.float32),
            pltpu.VMEM((_PER,), jnp.float32),
            pltpu.VMEM((_PER,), jnp.float32),
            pltpu.VMEM((_PER,), jnp.float32),
            pltpu.VMEM((_PER,), jnp.float32),
            pltpu.SemaphoreType.DMA,
        ],
    )(_box_body)
    out = f(hw_flat, idx_flat)
    return out.reshape(8, 4, _K).transpose(0, 2, 1)


def kernel(preds):
    B = preds.shape[0]
    scores, idx = _run_topk(preds)
    boxes = _run_boxes(preds, idx)
    return boxes, scores.reshape(B, _K)


# speedup vs baseline: 31.4310x; 1.5489x over previous
import functools

import jax
import jax.numpy as jnp
from jax import lax
from jax.experimental import pallas as pl
from jax.experimental.pallas import tpu as pltpu
from jax.experimental.pallas import tpu_sc as plsc

_H = 512
_W = 512
_R = 1024
_C = 256
_K = 1024
_N = _H * _W


_roll = pltpu.roll


def _dyn_cmpex(s, i, j, desc):
    r = lax.broadcasted_iota(jnp.int32, s.shape, 0)
    is_lo = (r & j) == 0
    ps = jnp.where(is_lo, _roll(s, -j, 0), _roll(s, j, 0))
    pi = jnp.where(is_lo, _roll(i, -j, 0), _roll(i, j, 0))
    beats = (s > ps) | ((s == ps) & (i < pi))
    keep = is_lo ^ beats ^ desc
    return jnp.where(keep, s, ps), jnp.where(keep, i, pi)


def _coldesc(shape, w4):
    if w4 == 0:
        return lax.broadcasted_iota(jnp.int32, shape, 1) >= 0
    c = lax.broadcasted_iota(jnp.int32, shape, 1)
    return (c & w4) == 0


def _sort_cols(s, i, coldesc):
    r = lax.broadcasted_iota(jnp.int32, s.shape, 0)

    def phase(p, si):
        kk = jnp.left_shift(jnp.int32(1), p)
        desc = ~(((r & kk) == 0) ^ coldesc)

        def stage(t, si):
            j = lax.shift_right_logical(kk, t + 1)
            return _dyn_cmpex(si[0], si[1], j, desc)

        return lax.fori_loop(0, p, stage, si)

    return lax.fori_loop(1, 11, phase, (s, i))


def _lane_cmpex(s, i, j, desc):
    ln = s.shape[1]
    l = lax.broadcasted_iota(jnp.int32, s.shape, 1)
    is_lo = (l & j) == 0
    ps = jnp.where(is_lo, _roll(s, ln - j, 1), _roll(s, j, 1))
    pi = jnp.where(is_lo, _roll(i, ln - j, 1), _roll(i, j, 1))
    beats = (s > ps) | ((s == ps) & (i < pi))
    keep = is_lo ^ beats ^ desc
    return jnp.where(keep, s, ps), jnp.where(keep, i, pi)


def _merge_tree(s, i):
    s, i = s.T, i.T
    while s.shape[0] > 1:
        h = s.shape[0] // 2
        sa, sb = s[:h], s[h:]
        ia, ib = i[:h], i[h:]
        beats = (sa > sb) | ((sa == sb) & (ia < ib))
        s = jnp.where(beats, sa, sb)
        i = jnp.where(beats, ia, ib)
        if h > 1:
            r = lax.broadcasted_iota(jnp.int32, s.shape, 0)
            desc = (r & (h // 2)) == 0
        else:
            desc = lax.broadcasted_iota(jnp.int32, s.shape, 0) >= 0
        j = _R // 2
        while j >= 1:
            s, i = _lane_cmpex(s, i, j, desc)
            j //= 2
    return s, i


def _topk_body(score_ref, scores_ref, idx_ref):
    s = score_ref[0, 0].reshape(_R, _C)
    idx = (lax.broadcasted_iota(jnp.int32, (_R, _C), 0) * _C
           + lax.broadcasted_iota(jnp.int32, (_R, _C), 1))
    s, idx = _sort_cols(s, idx, _coldesc((_R, _C), _C // 2))
    s1, i1 = _merge_tree(s, idx)
    scores_ref[0] = s1.reshape(8, 128)
    idx_ref[0] = i1.reshape(8, 128)


def _run_topk(preds):
    B = preds.shape[0]
    return pl.pallas_call(
        _topk_body,
        grid=(B,),
        in_specs=[pl.BlockSpec((1, 1, _H, _W), lambda b: (b, 0, 0, 0))],
        out_specs=[
            pl.BlockSpec((1, 8, 128), lambda b: (b, 0, 0)),
            pl.BlockSpec((1, 8, 128), lambda b: (b, 0, 0)),
        ],
        out_shape=[
            jax.ShapeDtypeStruct((B, 8, 128), jnp.float32),
            jax.ShapeDtypeStruct((B, 8, 128), jnp.int32),
        ],
    )(preds[:, :1])


_NW = 32
_PER = (8 * _K) // _NW


def _box_body(hw_hbm, idx_hbm, out_hbm, idxv, gh, gw, hv, wv, c0, c1, c2, c3,
              sem):
    wid = lax.axis_index("s") * 2 + lax.axis_index("c")
    b = wid // 4
    q = wid % 4
    base = wid * _PER

    pltpu.sync_copy(idx_hbm.at[pl.ds(base, _PER)], idxv)

    hbase = (b * 2) * _N
    wbase = (b * 2 + 1) * _N
    for k in range(_PER // 16):
        iv = idxv[pl.ds(k * 16, 16)]
        gh[k // 8, pl.ds((k % 8) * 16, 16)] = iv + hbase
        gw[k // 8, pl.ds((k % 8) * 16, 16)] = iv + wbase

    cps = []
    for j in range(2):
        cps.append(pltpu.async_copy(
            hw_hbm.at[gh.at[j]], hv.at[pl.ds(j * 128, 128)], sem))
        cps.append(pltpu.async_copy(
            hw_hbm.at[gw.at[j]], wv.at[pl.ds(j * 128, 128)], sem))
    for cp in cps:
        cp.wait()

    for k in range(_PER // 16):
        sl = pl.ds(k * 16, 16)
        iv = idxv[sl]
        xg = (iv & (_W - 1)).astype(jnp.float32)
        yg = (iv >> 9).astype(jnp.float32)
        hb = jnp.maximum(hv[sl], 1e-6) * _H
        wb = jnp.maximum(wv[sl], 1e-6) * _W
        c0[sl] = xg - wb * 0.5
        c1[sl] = yg - hb * 0.5
        c2[sl] = xg + wb * 0.5
        c3[sl] = yg + hb * 0.5

    obase = b * 4 * _K + q * _PER
    pltpu.sync_copy(c0, out_hbm.at[pl.ds(obase, _PER)])
    pltpu.sync_copy(c1, out_hbm.at[pl.ds(obase + _K, _PER)])
    pltpu.sync_copy(c2, out_hbm.at[pl.ds(obase + 2 * _K, _PER)])
    pltpu.sync_copy(c3, out_hbm.at[pl.ds(obase + 3 * _K, _PER)])


def _run_boxes(preds, idx):
    hw_flat = preds[:, 1:3].reshape(-1)
    idx_flat = idx.reshape(-1)
    mesh = plsc.VectorSubcoreMesh(core_axis_name="c", subcore_axis_name="s")
    f = functools.partial(
        pl.kernel,
        mesh=mesh,
        out_type=jax.ShapeDtypeStruct((8 * 4 * _K,), jnp.float32),
        scratch_types=[
            pltpu.VMEM((_PER,), jnp.int32),
            pltpu.VMEM((2, 128), jnp.int32),
            pltpu.VMEM((2, 128), jnp.int32),
            pltpu.VMEM((_PER,), jnp.float32),
            pltpu.VMEM((_PER,), jnp.float32),
            pltpu.VMEM((_PER,), jnp.float32),
            pltpu.VMEM((_PER,), jnp.float32),
            pltpu.VMEM((_PER,), jnp.float32),
            pltpu.VMEM((_PER,), jnp.float32),
            pltpu.SemaphoreType.DMA,
        ],
    )(_box_body)
    out = f(hw_flat, idx_flat)
    return out.reshape(8, 4, _K).transpose(0, 2, 1)


def kernel(preds):
    B = preds.shape[0]
    scores, idx = _run_topk(preds)
    boxes = _run_boxes(preds, idx)
    return boxes, scores.reshape(B, _K)


# speedup vs baseline: 45.1939x; 1.4379x over previous
import functools

import jax
import jax.numpy as jnp
from jax import lax
from jax.experimental import pallas as pl
from jax.experimental.pallas import tpu as pltpu
from jax.experimental.pallas import tpu_sc as plsc

_H = 512
_W = 512
_R = 1024
_C = 256
_K = 1024
_N = _H * _W


_roll = pltpu.roll


def _dyn_lane_cmpex(s, i, j, desc):
    ln = s.shape[1]
    l = lax.broadcasted_iota(jnp.int32, s.shape, 1)
    is_lo = (l & j) == 0
    ps = jnp.where(is_lo, _roll(s, ln - j, 1), _roll(s, j, 1))
    pi = jnp.where(is_lo, _roll(i, ln - j, 1), _roll(i, j, 1))
    beats = (s > ps) | ((s == ps) & (i < pi))
    keep = is_lo ^ beats ^ desc
    return jnp.where(keep, s, ps), jnp.where(keep, i, pi)


def _sort_rows(s, i):
    r = lax.broadcasted_iota(jnp.int32, s.shape, 0)
    l = lax.broadcasted_iota(jnp.int32, s.shape, 1)
    rowdesc = (r & (s.shape[0] // 2)) == 0

    def phase(p, si):
        kk = jnp.left_shift(jnp.int32(1), p)
        desc = ~(((l & kk) == 0) ^ rowdesc)

        def stage(t, si):
            j = lax.shift_right_logical(kk, t + 1)
            return _dyn_lane_cmpex(si[0], si[1], j, desc)

        return lax.fori_loop(0, p, stage, si)

    return lax.fori_loop(1, 11, phase, (s, i))


def _lane_cmpex(s, i, j, desc):
    ln = s.shape[1]
    l = lax.broadcasted_iota(jnp.int32, s.shape, 1)
    is_lo = (l & j) == 0
    ps = jnp.where(is_lo, _roll(s, ln - j, 1), _roll(s, j, 1))
    pi = jnp.where(is_lo, _roll(i, ln - j, 1), _roll(i, j, 1))
    beats = (s > ps) | ((s == ps) & (i < pi))
    keep = is_lo ^ beats ^ desc
    return jnp.where(keep, s, ps), jnp.where(keep, i, pi)


def _merge_tree(s, i):
    while s.shape[0] > 1:
        h = s.shape[0] // 2
        sa, sb = s[:h], s[h:]
        ia, ib = i[:h], i[h:]
        beats = (sa > sb) | ((sa == sb) & (ia < ib))
        s = jnp.where(beats, sa, sb)
        i = jnp.where(beats, ia, ib)
        if h > 1:
            r = lax.broadcasted_iota(jnp.int32, s.shape, 0)
            desc = (r & (h // 2)) == 0
        else:
            desc = lax.broadcasted_iota(jnp.int32, s.shape, 0) >= 0
        j = _R // 2
        while j >= 1:
            s, i = _lane_cmpex(s, i, j, desc)
            j //= 2
    return s, i


def _topk_body(score_ref, scores_ref, idx_ref):
    s = score_ref[0, 0].reshape(_C, _R)
    idx = (lax.broadcasted_iota(jnp.int32, (_C, _R), 0) * _R
           + lax.broadcasted_iota(jnp.int32, (_C, _R), 1))
    s, idx = _sort_rows(s, idx)
    s1, i1 = _merge_tree(s, idx)
    scores_ref[0] = s1.reshape(8, 128)
    idx_ref[0] = i1.reshape(8, 128)


def _run_topk(preds):
    B = preds.shape[0]
    return pl.pallas_call(
        _topk_body,
        grid=(B,),
        in_specs=[pl.BlockSpec((1, 1, _H, _W), lambda b: (b, 0, 0, 0))],
        out_specs=[
            pl.BlockSpec((1, 8, 128), lambda b: (b, 0, 0)),
            pl.BlockSpec((1, 8, 128), lambda b: (b, 0, 0)),
        ],
        out_shape=[
            jax.ShapeDtypeStruct((B, 8, 128), jnp.float32),
            jax.ShapeDtypeStruct((B, 8, 128), jnp.int32),
        ],
    )(preds[:, :1])


_NW = 32
_PER = (8 * _K) // _NW


def _box_body(hw_hbm, idx_hbm, out_hbm, idxv, gh, gw, hv, wv, c0, c1, c2, c3,
              sem):
    wid = lax.axis_index("s") * 2 + lax.axis_index("c")
    b = wid // 4
    q = wid % 4
    base = wid * _PER

    pltpu.sync_copy(idx_hbm.at[pl.ds(base, _PER)], idxv)

    hbase = (b * 2) * _N
    wbase = (b * 2 + 1) * _N
    for k in range(_PER // 16):
        iv = idxv[pl.ds(k * 16, 16)]
        gh[k // 8, pl.ds((k % 8) * 16, 16)] = iv + hbase
        gw[k // 8, pl.ds((k % 8) * 16, 16)] = iv + wbase

    cps = []
    for j in range(2):
        cps.append(pltpu.async_copy(
            hw_hbm.at[gh.at[j]], hv.at[pl.ds(j * 128, 128)], sem))
        cps.append(pltpu.async_copy(
            hw_hbm.at[gw.at[j]], wv.at[pl.ds(j * 128, 128)], sem))
    for cp in cps:
        cp.wait()

    for k in range(_PER // 16):
        sl = pl.ds(k * 16, 16)
        iv = idxv[sl]
        xg = (iv & (_W - 1)).astype(jnp.float32)
        yg = (iv >> 9).astype(jnp.float32)
        hb = jnp.maximum(hv[sl], 1e-6) * _H
        wb = jnp.maximum(wv[sl], 1e-6) * _W
        c0[sl] = xg - wb * 0.5
        c1[sl] = yg - hb * 0.5
        c2[sl] = xg + wb * 0.5
        c3[sl] = yg + hb * 0.5

    obase = b * 4 * _K + q * _PER
    pltpu.sync_copy(c0, out_hbm.at[pl.ds(obase, _PER)])
    pltpu.sync_copy(c1, out_hbm.at[pl.ds(obase + _K, _PER)])
    pltpu.sync_copy(c2, out_hbm.at[pl.ds(obase + 2 * _K, _PER)])
    pltpu.sync_copy(c3, out_hbm.at[pl.ds(obase + 3 * _K, _PER)])


def _run_boxes(preds, idx):
    hw_flat = preds[:, 1:3].reshape(-1)
    idx_flat = idx.reshape(-1)
    mesh = plsc.VectorSubcoreMesh(core_axis_name="c", subcore_axis_name="s")
    f = functools.partial(
        pl.kernel,
        mesh=mesh,
        out_type=jax.ShapeDtypeStruct((8 * 4 * _K,), jnp.float32),
        scratch_types=[
            pltpu.VMEM((_PER,), jnp.int32),
            pltpu.VMEM((2, 128), jnp.int32),
            pltpu.VMEM((2, 128), jnp.int32),
            pltpu.VMEM((_PER,), jnp.float32),
            pltpu.VMEM((_PER,), jnp.float32),
            pltpu.VMEM((_PER,), jnp.float32),
            pltpu.VMEM((_PER,), jnp.float32),
            pltpu.VMEM((_PER,), jnp.float32),
            pltpu.VMEM((_PER,), jnp.float32),
            pltpu.SemaphoreType.DMA,
        ],
    )(_box_body)
    out = f(hw_flat, idx_flat)
    return out.reshape(8, 4, _K).transpose(0, 2, 1)


def kernel(preds):
    B = preds.shape[0]
    scores, idx = _run_topk(preds)
    boxes = _run_boxes(preds, idx)
    return boxes, scores.reshape(B, _K)


# speedup vs baseline: 57.1134x; 1.2637x over previous
import functools

import jax
import jax.numpy as jnp
from jax import lax
from jax.experimental import pallas as pl
from jax.experimental.pallas import tpu as pltpu
from jax.experimental.pallas import tpu_sc as plsc

_H = 512
_W = 512
_R = 1024
_C = 256
_K = 1024
_N = _H * _W


_roll = pltpu.roll


def _dyn_lane_cmpex(s, i, j, desc):
    ln = s.shape[1]
    l = lax.broadcasted_iota(jnp.int32, s.shape, 1)
    is_lo = (l & j) == 0
    ps = jnp.where(is_lo, _roll(s, ln - j, 1), _roll(s, j, 1))
    pi = jnp.where(is_lo, _roll(i, ln - j, 1), _roll(i, j, 1))
    beats = (s > ps) | ((s == ps) & (i < pi))
    keep = is_lo ^ beats ^ desc
    return jnp.where(keep, s, ps), jnp.where(keep, i, pi)


def _sort_rows(s, i):
    r = lax.broadcasted_iota(jnp.int32, s.shape, 0)
    l = lax.broadcasted_iota(jnp.int32, s.shape, 1)
    rowdesc = (r & (s.shape[0] // 2)) == 0

    def stage(g, si):
        p = jnp.int32(1)
        for k in (1, 3, 6, 10, 15, 21, 28, 36, 45):
            p = p + jnp.where(g >= k, 1, 0).astype(jnp.int32)
        start = (p * (p - 1)) // 2
        kk = jnp.left_shift(jnp.int32(1), p)
        j = jnp.left_shift(jnp.int32(1), p - 1 - (g - start))
        desc = ~(((l & kk) == 0) ^ rowdesc)
        return _dyn_lane_cmpex(si[0], si[1], j, desc)

    return lax.fori_loop(0, 55, stage, (s, i), unroll=2)


def _lane_cmpex(s, i, j, desc):
    ln = s.shape[1]
    l = lax.broadcasted_iota(jnp.int32, s.shape, 1)
    is_lo = (l & j) == 0
    ps = jnp.where(is_lo, _roll(s, ln - j, 1), _roll(s, j, 1))
    pi = jnp.where(is_lo, _roll(i, ln - j, 1), _roll(i, j, 1))
    beats = (s > ps) | ((s == ps) & (i < pi))
    keep = is_lo ^ beats ^ desc
    return jnp.where(keep, s, ps), jnp.where(keep, i, pi)


def _merge_tree(s, i):
    while s.shape[0] > 1:
        h = s.shape[0] // 2
        sa, sb = s[:h], s[h:]
        ia, ib = i[:h], i[h:]
        beats = (sa > sb) | ((sa == sb) & (ia < ib))
        s = jnp.where(beats, sa, sb)
        i = jnp.where(beats, ia, ib)
        if h > 1:
            r = lax.broadcasted_iota(jnp.int32, s.shape, 0)
            desc = (r & (h // 2)) == 0
        else:
            desc = lax.broadcasted_iota(jnp.int32, s.shape, 0) >= 0
        j = _R // 2
        while j >= 1:
            s, i = _lane_cmpex(s, i, j, desc)
            j //= 2
    return s, i


def _topk_body(score_ref, scores_ref, idx_ref):
    s = score_ref[0, 0].reshape(_C, _R)
    idx = (lax.broadcasted_iota(jnp.int32, (_C, _R), 0) * _R
           + lax.broadcasted_iota(jnp.int32, (_C, _R), 1))
    s, idx = _sort_rows(s, idx)
    s1, i1 = _merge_tree(s, idx)
    scores_ref[0] = s1.reshape(8, 128)
    idx_ref[0] = i1.reshape(8, 128)


def _run_topk(preds):
    B = preds.shape[0]
    return pl.pallas_call(
        _topk_body,
        grid=(B,),
        in_specs=[pl.BlockSpec((1, 1, _H, _W), lambda b: (b, 0, 0, 0))],
        out_specs=[
            pl.BlockSpec((1, 8, 128), lambda b: (b, 0, 0)),
            pl.BlockSpec((1, 8, 128), lambda b: (b, 0, 0)),
        ],
        out_shape=[
            jax.ShapeDtypeStruct((B, 8, 128), jnp.float32),
            jax.ShapeDtypeStruct((B, 8, 128), jnp.int32),
        ],
    )(preds[:, :1])


_NW = 32
_PER = (8 * _K) // _NW


def _box_body(hw_hbm, idx_hbm, out_hbm, idxv, gh, gw, hv, wv, c0, c1, c2, c3,
              sem):
    wid = lax.axis_index("s") * 2 + lax.axis_index("c")
    b = wid // 4
    q = wid % 4
    base = wid * _PER

    pltpu.sync_copy(idx_hbm.at[pl.ds(base, _PER)], idxv)

    hbase = (b * 2) * _N
    wbase = (b * 2 + 1) * _N
    for k in range(_PER // 16):
        iv = idxv[pl.ds(k * 16, 16)]
        gh[k // 8, pl.ds((k % 8) * 16, 16)] = iv + hbase
        gw[k // 8, pl.ds((k % 8) * 16, 16)] = iv + wbase

    cps = []
    for j in range(2):
        cps.append(pltpu.async_copy(
            hw_hbm.at[gh.at[j]], hv.at[pl.ds(j * 128, 128)], sem))
        cps.append(pltpu.async_copy(
            hw_hbm.at[gw.at[j]], wv.at[pl.ds(j * 128, 128)], sem))
    for cp in cps:
        cp.wait()

    for k in range(_PER // 16):
        sl = pl.ds(k * 16, 16)
        iv = idxv[sl]
        xg = (iv & (_W - 1)).astype(jnp.float32)
        yg = (iv >> 9).astype(jnp.float32)
        hb = jnp.maximum(hv[sl], 1e-6) * _H
        wb = jnp.maximum(wv[sl], 1e-6) * _W
        c0[sl] = xg - wb * 0.5
        c1[sl] = yg - hb * 0.5
        c2[sl] = xg + wb * 0.5
        c3[sl] = yg + hb * 0.5

    obase = b * 4 * _K + q * _PER
    pltpu.sync_copy(c0, out_hbm.at[pl.ds(obase, _PER)])
    pltpu.sync_copy(c1, out_hbm.at[pl.ds(obase + _K, _PER)])
    pltpu.sync_copy(c2, out_hbm.at[pl.ds(obase + 2 * _K, _PER)])
    pltpu.sync_copy(c3, out_hbm.at[pl.ds(obase + 3 * _K, _PER)])


def _run_boxes(preds, idx):
    hw_flat = preds[:, 1:3].reshape(-1)
    idx_flat = idx.reshape(-1)
    mesh = plsc.VectorSubcoreMesh(core_axis_name="c", subcore_axis_name="s")
    f = functools.partial(
        pl.kernel,
        mesh=mesh,
        out_type=jax.ShapeDtypeStruct((8 * 4 * _K,), jnp.float32),
        scratch_types=[
            pltpu.VMEM((_PER,), jnp.int32),
            pltpu.VMEM((2, 128), jnp.int32),
            pltpu.VMEM((2, 128), jnp.int32),
            pltpu.VMEM((_PER,), jnp.float32),
            pltpu.VMEM((_PER,), jnp.float32),
            pltpu.VMEM((_PER,), jnp.float32),
            pltpu.VMEM((_PER,), jnp.float32),
            pltpu.VMEM((_PER,), jnp.float32),
            pltpu.VMEM((_PER,), jnp.float32),
            pltpu.SemaphoreType.DMA,
        ],
    )(_box_body)
    out = f(hw_flat, idx_flat)
    return out.reshape(8, 4, _K).transpose(0, 2, 1)


def kernel(preds):
    B = preds.shape[0]
    scores, idx = _run_topk(preds)
    boxes = _run_boxes(preds, idx)
    return boxes, scores.reshape(B, _K)
